# Initial kernel scaffold; baseline (speedup 1.0000x reference)
import jax
import jax.numpy as jnp
from jax.experimental import pallas as pl


def kernel(x, conv_w, conv_b, gamma, beta, fc1_w, fc1_b, fc2_w, fc2_b):
    raise NotImplementedError("write your pallas kernel here")


# speedup vs baseline: 5.6991x; 5.6991x over previous
import numpy as np

import jax
import jax.numpy as jnp
from jax import lax
from jax.experimental import pallas as pl
from jax.experimental.pallas import tpu as pltpu

_EPS = 1e-5
_TM = 64
_VMEM_LIMIT = 48 * 1024 * 1024


def _round_up(x, k):
    return (x + k - 1) // k * k


def _build_x3(xt, tm, H, W):
    z = jnp.zeros((tm, 1, W), jnp.float32)
    xv = jnp.concatenate([z, xt, z], axis=1)
    xr = xv.reshape(tm, (H + 2) // 2, 2, W)
    ev = xr[:, :, 0, :]
    od = xr[:, :, 1, :]
    h2 = H // 2

    def sec(a, b):
        return jnp.concatenate([a[:, None], b[:, None]], axis=1).reshape(tm * H, W)

    s0 = sec(ev[:, 0:h2], od[:, 0:h2])
    s1 = sec(od[:, 0:h2], ev[:, 1:h2 + 1])
    s2 = sec(ev[:, 1:h2 + 1], od[:, 1:h2 + 1])
    ones = jnp.ones((tm * H, 1), jnp.float32)
    return jnp.concatenate([s0, s1, s2, ones], axis=1)


def _gram_kernel(xt_ref, out_ref):
    tm, H, W = xt_ref.shape
    x3 = _build_x3(xt_ref[...], tm, H, W)
    gg = lax.dot_general(x3, x3, (((0,), (0,)), ((), ())),
                         preferred_element_type=jnp.float32)
    out_ref[...] = gg[None]


def _main_kernel(xt_ref, b_ref, fc1_ref, vec_ref, out_ref):
    tm, H, W = xt_ref.shape
    C = fc1_ref.shape[0]
    h2, w2 = H // 2, W // 2
    x3 = _build_x3(xt_ref[...], tm, H, W)
    y = jnp.dot(x3, b_ref[...], preferred_element_type=jnp.float32)
    y = y.reshape(tm, 2, h2, W * C)
    v = jnp.maximum(y[:, 0], y[:, 1]).reshape(tm * h2, W * C)
    half = w2 * C
    z = jnp.maximum(jnp.maximum(v[:, :half], v[:, half:]), 0.0)
    s = z[:, 0:C]
    for k in range(1, w2):
        s = s + z[:, k * C:(k + 1) * C]
    feat = jnp.sum(s.reshape(tm, h2, C), axis=1)
    vecs = vec_ref[...]
    h = jnp.dot(feat, fc1_ref[...], preferred_element_type=jnp.float32) + vecs[0:1, :]
    h = jnp.maximum(h, 0.0)
    logit = jnp.sum(h * vecs[1:2, :], axis=-1, keepdims=True) + vecs[2:3, 0:1]
    out_ref[...] = (1.0 / (1.0 + jnp.exp(-logit))).reshape(1, tm, 1)


def kernel(x, conv_w, conv_b, gamma, beta, fc1_w, fc1_b, fc2_w, fc2_b):
    d0, d1, J, H, W = x.shape
    assert H % 2 == 0 and W % 2 == 0
    M = d0 * d1 * J
    C = conv_w.shape[-1]
    K3 = 3 * W + 1

    xm = x.reshape(M, H, W).astype(jnp.float32)
    w9 = conv_w.reshape(9, C).astype(jnp.float32)

    tm = min(_TM, _round_up(M, 8))
    Mp = _round_up(M, tm)
    nt = Mp // tm
    xp = jnp.pad(xm, ((0, Mp - M), (0, 0), (0, 0)))

    stats = pl.pallas_call(
        _gram_kernel,
        out_shape=jax.ShapeDtypeStruct((nt, K3, K3), jnp.float32),
        grid=(nt,),
        in_specs=[pl.BlockSpec((tm, H, W), lambda i: (i, 0, 0))],
        out_specs=pl.BlockSpec((1, K3, K3), lambda i: (i, 0, 0)),
        compiler_params=pltpu.CompilerParams(
            dimension_semantics=("parallel",),
            vmem_limit_bytes=_VMEM_LIMIT),
    )(xp)
    GG = jnp.sum(stats, axis=0)
    colsum = GG[K3 - 1, :K3 - 1]

    idxS = np.zeros((9, W), np.int32)
    mskS = np.zeros((9, W), np.float32)
    for ky in range(3):
        for kx in range(3):
            k = ky * 3 + kx
            for w in range(W):
                wp = w + kx - 1
                if 0 <= wp < W:
                    idxS[k, w] = ky * W + wp
                    mskS[k, w] = 1.0
    S = jnp.sum(colsum[idxS] * mskS, axis=1)

    idxR = np.zeros((81, W), np.int32)
    idxC = np.zeros((81, W), np.int32)
    mskG = np.zeros((81, W), np.float32)
    for k in range(9):
        ky_k, kx_k = divmod(k, 3)
        for l in range(9):
            ky_l, kx_l = divmod(l, 3)
            for w in range(W):
                wk, wl = w + kx_k - 1, w + kx_l - 1
                if 0 <= wk < W and 0 <= wl < W:
                    idxR[k * 9 + l, w] = ky_k * W + wk
                    idxC[k * 9 + l, w] = ky_l * W + wl
                    mskG[k * 9 + l, w] = 1.0
    G = jnp.sum(GG[idxR, idxC] * mskG, axis=1).reshape(9, 9)

    count = float(M * H * W)
    mean = jnp.dot(S, w9) / count
    ssq = jnp.einsum("kc,kl,lc->c", w9, G, w9)
    var = jnp.maximum(ssq / count - mean * mean, 0.0)
    scale = gamma * lax.rsqrt(var + _EPS)
    shift = beta - scale * mean
    pool_inv = 1.0 / ((H // 2) * (W // 2))
    sf = scale * pool_inv
    hf = shift * pool_inv

    rows_l, cbase_l, taps_l = [], [], []
    for ky in range(3):
        for kx in range(3):
            for w in range(W):
                wp = w + kx - 1
                if 0 <= wp < W:
                    rows_l.append(ky * W + wp)
                    cbase_l.append((w % 2) * (W // 2) * C + (w // 2) * C)
                    taps_l.append(ky * 3 + kx)
    E = len(rows_l)
    rows_f = np.repeat(np.array(rows_l, np.int32), C)
    cols_f = np.repeat(np.array(cbase_l, np.int32), C) + np.tile(np.arange(C, dtype=np.int32), E)
    w9s = w9 * sf[None, :]
    vals = w9s[np.array(taps_l, np.int32)].reshape(-1)
    B = jnp.zeros((K3, W * C), jnp.float32).at[rows_f, cols_f].set(vals)
    B = B.at[K3 - 1, :].set(jnp.tile(hf, W))

    vecs = jnp.stack([fc1_b, fc2_w.reshape(-1),
                      jnp.full((C,), fc2_b[0], jnp.float32)], axis=0)

    scores = pl.pallas_call(
        _main_kernel,
        out_shape=jax.ShapeDtypeStruct((nt, tm, 1), jnp.float32),
        grid=(nt,),
        in_specs=[pl.BlockSpec((tm, H, W), lambda i: (i, 0, 0)),
                  pl.BlockSpec((K3, W * C), lambda i: (0, 0)),
                  pl.BlockSpec((C, C), lambda i: (0, 0)),
                  pl.BlockSpec((3, C), lambda i: (0, 0))],
        out_specs=pl.BlockSpec((1, tm, 1), lambda i: (i, 0, 0)),
        compiler_params=pltpu.CompilerParams(
            dimension_semantics=("parallel",),
            vmem_limit_bytes=_VMEM_LIMIT),
    )(xp, B, fc1_w, vecs)

    return scores.reshape(Mp, 1)[:M].reshape(d0 * d1, J, 1)


# speedup vs baseline: 9.1390x; 1.6036x over previous
import numpy as np

import jax
import jax.numpy as jnp
from jax import lax
from jax.experimental import pallas as pl
from jax.experimental.pallas import tpu as pltpu

_EPS = 1e-5
_TM = 64
_VMEM_LIMIT = 48 * 1024 * 1024


def _round_up(x, k):
    return (x + k - 1) // k * k


def _build_x3(xt, tm, H, W):
    h2 = H // 2
    xe = xt[:, 0]
    xo = xt[:, 1]
    z = jnp.zeros((tm, 1, W), jnp.float32)
    dn = jnp.concatenate([z, xo[:, :h2 - 1]], axis=1)
    up = jnp.concatenate([xe[:, 1:], z], axis=1)

    def sec(a, b):
        return jnp.concatenate([a[:, None], b[:, None]], axis=1).reshape(tm * H, W)

    s0 = sec(dn, xe)
    s1 = xt.reshape(tm * H, W)
    s2 = sec(xo, up)
    ones = jnp.ones((tm * H, 1), jnp.float32)
    return jnp.concatenate([s0, s1, s2, ones], axis=1)


def _gram_kernel(xt_ref, out_ref):
    tm, _, h2, W = xt_ref.shape
    H = 2 * h2
    x3 = _build_x3(xt_ref[...], tm, H, W)
    gg = lax.dot_general(x3, x3, (((0,), (0,)), ((), ())),
                         preferred_element_type=jnp.float32)
    out_ref[...] = gg[None]


def _main_kernel(xt_ref, b_ref, fc1_ref, vec_ref, out_ref):
    tm, _, h2, W = xt_ref.shape
    H = 2 * h2
    C = fc1_ref.shape[0]
    w2 = W // 2
    x3 = _build_x3(xt_ref[...], tm, H, W)
    y = jnp.dot(x3, b_ref[...], preferred_element_type=jnp.float32)
    y = y.reshape(tm, 2, h2, W * C)
    v = jnp.maximum(y[:, 0], y[:, 1]).reshape(tm * h2, W * C)
    half = w2 * C
    z = jnp.maximum(jnp.maximum(v[:, :half], v[:, half:]), 0.0)
    while z.shape[1] > C:
        hw = z.shape[1] // 2
        z = z[:, :hw] + z[:, hw:]
    feat = jnp.sum(z.reshape(tm, h2, C), axis=1)
    vecs = vec_ref[...]
    h = jnp.dot(feat, fc1_ref[...], preferred_element_type=jnp.float32) + vecs[0:1, :]
    h = jnp.maximum(h, 0.0)
    logit = jnp.sum(h * vecs[1:2, :], axis=-1, keepdims=True) + vecs[2:3, 0:1]
    out_ref[...] = (1.0 / (1.0 + jnp.exp(-logit))).reshape(1, tm, 1)


def kernel(x, conv_w, conv_b, gamma, beta, fc1_w, fc1_b, fc2_w, fc2_b):
    d0, d1, J, H, W = x.shape
    assert H % 2 == 0 and W % 2 == 0
    M = d0 * d1 * J
    C = conv_w.shape[-1]
    K3 = 3 * W + 1

    xm = x.reshape(M, H, W).astype(jnp.float32)
    w9 = conv_w.reshape(9, C).astype(jnp.float32)

    tm = min(_TM, _round_up(M, 8))
    Mp = _round_up(M, tm)
    nt = Mp // tm
    xp = jnp.pad(xm, ((0, Mp - M), (0, 0), (0, 0)))
    xr = xp.reshape(Mp, H // 2, 2, W).transpose(0, 2, 1, 3)

    stats = pl.pallas_call(
        _gram_kernel,
        out_shape=jax.ShapeDtypeStruct((nt, K3, K3), jnp.float32),
        grid=(nt,),
        in_specs=[pl.BlockSpec((tm, 2, H // 2, W), lambda i: (i, 0, 0, 0))],
        out_specs=pl.BlockSpec((1, K3, K3), lambda i: (i, 0, 0)),
        compiler_params=pltpu.CompilerParams(
            dimension_semantics=("parallel",),
            vmem_limit_bytes=_VMEM_LIMIT),
    )(xr)
    GG = jnp.sum(stats, axis=0)
    colsum = GG[K3 - 1, :K3 - 1]

    idxS = np.zeros((9, W), np.int32)
    mskS = np.zeros((9, W), np.float32)
    for ky in range(3):
        for kx in range(3):
            k = ky * 3 + kx
            for w in range(W):
                wp = w + kx - 1
                if 0 <= wp < W:
                    idxS[k, w] = ky * W + wp
                    mskS[k, w] = 1.0
    S = jnp.sum(colsum[idxS] * mskS, axis=1)

    idxR = np.zeros((81, W), np.int32)
    idxC = np.zeros((81, W), np.int32)
    mskG = np.zeros((81, W), np.float32)
    for k in range(9):
        ky_k, kx_k = divmod(k, 3)
        for l in range(9):
            ky_l, kx_l = divmod(l, 3)
            for w in range(W):
                wk, wl = w + kx_k - 1, w + kx_l - 1
                if 0 <= wk < W and 0 <= wl < W:
                    idxR[k * 9 + l, w] = ky_k * W + wk
                    idxC[k * 9 + l, w] = ky_l * W + wl
                    mskG[k * 9 + l, w] = 1.0
    G = jnp.sum(GG[idxR, idxC] * mskG, axis=1).reshape(9, 9)

    count = float(M * H * W)
    mean = jnp.dot(S, w9) / count
    ssq = jnp.einsum("kc,kl,lc->c", w9, G, w9)
    var = jnp.maximum(ssq / count - mean * mean, 0.0)
    scale = gamma * lax.rsqrt(var + _EPS)
    shift = beta - scale * mean
    pool_inv = 1.0 / ((H // 2) * (W // 2))
    sf = scale * pool_inv
    hf = shift * pool_inv

    rows_l, cbase_l, taps_l = [], [], []
    for ky in range(3):
        for kx in range(3):
            for w in range(W):
                wp = w + kx - 1
                if 0 <= wp < W:
                    rows_l.append(ky * W + wp)
                    cbase_l.append((w % 2) * (W // 2) * C + (w // 2) * C)
                    taps_l.append(ky * 3 + kx)
    E = len(rows_l)
    rows_f = np.repeat(np.array(rows_l, np.int32), C)
    cols_f = np.repeat(np.array(cbase_l, np.int32), C) + np.tile(np.arange(C, dtype=np.int32), E)
    w9s = w9 * sf[None, :]
    vals = w9s[np.array(taps_l, np.int32)].reshape(-1)
    B = jnp.zeros((K3, W * C), jnp.float32).at[rows_f, cols_f].set(vals)
    B = B.at[K3 - 1, :].set(jnp.tile(hf, W))

    vecs = jnp.stack([fc1_b, fc2_w.reshape(-1),
                      jnp.full((C,), fc2_b[0], jnp.float32)], axis=0)

    scores = pl.pallas_call(
        _main_kernel,
        out_shape=jax.ShapeDtypeStruct((nt, tm, 1), jnp.float32),
        grid=(nt,),
        in_specs=[pl.BlockSpec((tm, 2, H // 2, W), lambda i: (i, 0, 0, 0)),
                  pl.BlockSpec((K3, W * C), lambda i: (0, 0)),
                  pl.BlockSpec((C, C), lambda i: (0, 0)),
                  pl.BlockSpec((3, C), lambda i: (0, 0))],
        out_specs=pl.BlockSpec((1, tm, 1), lambda i: (i, 0, 0)),
        compiler_params=pltpu.CompilerParams(
            dimension_semantics=("parallel",),
            vmem_limit_bytes=_VMEM_LIMIT),
    )(xr, B, fc1_w, vecs)

    return scores.reshape(Mp, 1)[:M].reshape(d0 * d1, J, 1)


# speedup vs baseline: 14.7045x; 1.6090x over previous
import numpy as np

import jax
import jax.numpy as jnp
from jax import lax
from jax.experimental import pallas as pl
from jax.experimental.pallas import tpu as pltpu

_EPS = 1e-5
_TM = 256
_VMEM_LIMIT = 48 * 1024 * 1024


def _round_up(x, k):
    return (x + k - 1) // k * k


def _build_x3(xt, tm, H, W):
    h2 = H // 2
    xe = xt[:, 0]
    xo = xt[:, 1]
    z = jnp.zeros((tm, 1, W), jnp.float32)
    dn = jnp.concatenate([z, xo[:, :h2 - 1]], axis=1)
    up = jnp.concatenate([xe[:, 1:], z], axis=1)

    def sec(a, b):
        return jnp.concatenate([a[:, None], b[:, None]], axis=1).reshape(tm * H, W)

    s0 = sec(dn, xe)
    s1 = xt.reshape(tm * H, W)
    s2 = sec(xo, up)
    ones = jnp.ones((tm * H, 1), jnp.float32)
    return jnp.concatenate([s0, s1, s2, ones], axis=1)


def _gram_kernel(xt_ref, out_ref):
    tm, _, h2, W = xt_ref.shape
    H = 2 * h2
    x3 = _build_x3(xt_ref[...], tm, H, W)
    gg = lax.dot_general(x3, x3, (((0,), (0,)), ((), ())),
                         preferred_element_type=jnp.float32)
    out_ref[...] = gg[None]


def _main_kernel(xt_ref, b_ref, fc1_ref, vec_ref, out_ref):
    tm, _, h2, W = xt_ref.shape
    H = 2 * h2
    C = fc1_ref.shape[0]
    w2 = W // 2
    x3 = _build_x3(xt_ref[...], tm, H, W)
    y = jnp.dot(x3, b_ref[...], preferred_element_type=jnp.float32)
    y = y.reshape(tm, 2, h2, W * C)
    v = jnp.maximum(y[:, 0], y[:, 1]).reshape(tm * h2, W * C)
    half = w2 * C
    z = jnp.maximum(jnp.maximum(v[:, :half], v[:, half:]), 0.0)
    while z.shape[1] > C:
        hw = z.shape[1] // 2
        z = z[:, :hw] + z[:, hw:]
    feat = jnp.sum(z.reshape(tm, h2, C), axis=1)
    vecs = vec_ref[...]
    h = jnp.dot(feat, fc1_ref[...], preferred_element_type=jnp.float32) + vecs[0:1, :]
    h = jnp.maximum(h, 0.0)
    logit = jnp.sum(h * vecs[1:2, :], axis=-1, keepdims=True) + vecs[2:3, 0:1]
    out_ref[...] = (1.0 / (1.0 + jnp.exp(-logit))).reshape(1, tm, 1)


def kernel(x, conv_w, conv_b, gamma, beta, fc1_w, fc1_b, fc2_w, fc2_b):
    d0, d1, J, H, W = x.shape
    assert H % 2 == 0 and W % 2 == 0
    M = d0 * d1 * J
    C = conv_w.shape[-1]
    K3 = 3 * W + 1

    xm = x.reshape(M, H, W).astype(jnp.float32)
    w9 = conv_w.reshape(9, C).astype(jnp.float32)

    tm = min(_TM, _round_up(M, 8))
    Mp = _round_up(M, tm)
    nt = Mp // tm
    xp = jnp.pad(xm, ((0, Mp - M), (0, 0), (0, 0)))
    xr = xp.reshape(Mp, H // 2, 2, W).transpose(0, 2, 1, 3)

    stats = pl.pallas_call(
        _gram_kernel,
        out_shape=jax.ShapeDtypeStruct((nt, K3, K3), jnp.float32),
        grid=(nt,),
        in_specs=[pl.BlockSpec((tm, 2, H // 2, W), lambda i: (i, 0, 0, 0))],
        out_specs=pl.BlockSpec((1, K3, K3), lambda i: (i, 0, 0)),
        compiler_params=pltpu.CompilerParams(
            dimension_semantics=("parallel",),
            vmem_limit_bytes=_VMEM_LIMIT),
    )(xr)
    GG = jnp.sum(stats, axis=0)
    colsum = GG[K3 - 1, :K3 - 1]

    idxS = np.zeros((9, W), np.int32)
    mskS = np.zeros((9, W), np.float32)
    for ky in range(3):
        for kx in range(3):
            k = ky * 3 + kx
            for w in range(W):
                wp = w + kx - 1
                if 0 <= wp < W:
                    idxS[k, w] = ky * W + wp
                    mskS[k, w] = 1.0
    S = jnp.sum(colsum[idxS] * mskS, axis=1)

    idxR = np.zeros((81, W), np.int32)
    idxC = np.zeros((81, W), np.int32)
    mskG = np.zeros((81, W), np.float32)
    for k in range(9):
        ky_k, kx_k = divmod(k, 3)
        for l in range(9):
            ky_l, kx_l = divmod(l, 3)
            for w in range(W):
                wk, wl = w + kx_k - 1, w + kx_l - 1
                if 0 <= wk < W and 0 <= wl < W:
                    idxR[k * 9 + l, w] = ky_k * W + wk
                    idxC[k * 9 + l, w] = ky_l * W + wl
                    mskG[k * 9 + l, w] = 1.0
    G = jnp.sum(GG[idxR, idxC] * mskG, axis=1).reshape(9, 9)

    count = float(M * H * W)
    mean = jnp.dot(S, w9) / count
    ssq = jnp.einsum("kc,kl,lc->c", w9, G, w9)
    var = jnp.maximum(ssq / count - mean * mean, 0.0)
    scale = gamma * lax.rsqrt(var + _EPS)
    shift = beta - scale * mean
    pool_inv = 1.0 / ((H // 2) * (W // 2))
    sf = scale * pool_inv
    hf = shift * pool_inv

    rows_l, cbase_l, taps_l = [], [], []
    for ky in range(3):
        for kx in range(3):
            for w in range(W):
                wp = w + kx - 1
                if 0 <= wp < W:
                    rows_l.append(ky * W + wp)
                    cbase_l.append((w % 2) * (W // 2) * C + (w // 2) * C)
                    taps_l.append(ky * 3 + kx)
    E = len(rows_l)
    rows_f = np.repeat(np.array(rows_l, np.int32), C)
    cols_f = np.repeat(np.array(cbase_l, np.int32), C) + np.tile(np.arange(C, dtype=np.int32), E)
    w9s = w9 * sf[None, :]
    vals = w9s[np.array(taps_l, np.int32)].reshape(-1)
    B = jnp.zeros((K3, W * C), jnp.float32).at[rows_f, cols_f].set(vals)
    B = B.at[K3 - 1, :].set(jnp.tile(hf, W))

    vecs = jnp.stack([fc1_b, fc2_w.reshape(-1),
                      jnp.full((C,), fc2_b[0], jnp.float32)], axis=0)

    scores = pl.pallas_call(
        _main_kernel,
        out_shape=jax.ShapeDtypeStruct((nt, tm, 1), jnp.float32),
        grid=(nt,),
        in_specs=[pl.BlockSpec((tm, 2, H // 2, W), lambda i: (i, 0, 0, 0)),
                  pl.BlockSpec((K3, W * C), lambda i: (0, 0)),
                  pl.BlockSpec((C, C), lambda i: (0, 0)),
                  pl.BlockSpec((3, C), lambda i: (0, 0))],
        out_specs=pl.BlockSpec((1, tm, 1), lambda i: (i, 0, 0)),
        compiler_params=pltpu.CompilerParams(
            dimension_semantics=("parallel",),
            vmem_limit_bytes=_VMEM_LIMIT),
    )(xr, B, fc1_w, vecs)

    return scores.reshape(Mp, 1)[:M].reshape(d0 * d1, J, 1)


# speedup vs baseline: 16.0761x; 1.0933x over previous
import numpy as np

import jax
import jax.numpy as jnp
from jax import lax
from jax.experimental import pallas as pl
from jax.experimental.pallas import tpu as pltpu

_EPS = 1e-5
_TM = 512
_VMEM_LIMIT = 60 * 1024 * 1024


def _round_up(x, k):
    return (x + k - 1) // k * k


def _build_x3(xt, tm, H, W):
    h2 = H // 2
    xe = xt[:, 0]
    xo = xt[:, 1]
    z = jnp.zeros((tm, 1, W), jnp.float32)
    dn = jnp.concatenate([z, xo[:, :h2 - 1]], axis=1)
    up = jnp.concatenate([xe[:, 1:], z], axis=1)

    def sec(a, b):
        return jnp.concatenate([a[:, None], b[:, None]], axis=1).reshape(tm * H, W)

    s0 = sec(dn, xe)
    s1 = xt.reshape(tm * H, W)
    s2 = sec(xo, up)
    ones = jnp.ones((tm * H, 1), jnp.float32)
    return jnp.concatenate([s0, s1, s2, ones], axis=1)


def _gram_kernel(xt_ref, out_ref):
    tm, _, h2, W = xt_ref.shape
    H = 2 * h2
    x3 = _build_x3(xt_ref[...], tm, H, W)
    gg = lax.dot_general(x3, x3, (((0,), (0,)), ((), ())),
                         preferred_element_type=jnp.float32)
    out_ref[...] = gg[None]


def _main_kernel(xt_ref, b_ref, fc1_ref, vec_ref, out_ref):
    tm, _, h2, W = xt_ref.shape
    H = 2 * h2
    C = fc1_ref.shape[0]
    w2 = W // 2
    x3 = _build_x3(xt_ref[...], tm, H, W)
    y = jnp.dot(x3, b_ref[...], preferred_element_type=jnp.float32)
    y = y.reshape(tm, 2, h2, W * C)
    v = jnp.maximum(y[:, 0], y[:, 1]).reshape(tm * h2, W * C)
    half = w2 * C
    z = jnp.maximum(jnp.maximum(v[:, :half], v[:, half:]), 0.0)
    while z.shape[1] > C:
        hw = z.shape[1] // 2
        z = z[:, :hw] + z[:, hw:]
    feat = jnp.sum(z.reshape(tm, h2, C), axis=1)
    vecs = vec_ref[...]
    h = jnp.dot(feat, fc1_ref[...], preferred_element_type=jnp.float32) + vecs[0:1, :]
    h = jnp.maximum(h, 0.0)
    logit = jnp.sum(h * vecs[1:2, :], axis=-1, keepdims=True) + vecs[2:3, 0:1]
    out_ref[...] = (1.0 / (1.0 + jnp.exp(-logit))).reshape(1, tm, 1)


def kernel(x, conv_w, conv_b, gamma, beta, fc1_w, fc1_b, fc2_w, fc2_b):
    d0, d1, J, H, W = x.shape
    assert H % 2 == 0 and W % 2 == 0
    M = d0 * d1 * J
    C = conv_w.shape[-1]
    K3 = 3 * W + 1

    xm = x.reshape(M, H, W).astype(jnp.float32)
    w9 = conv_w.reshape(9, C).astype(jnp.float32)

    tm = min(_TM, _round_up(M, 8))
    Mp = _round_up(M, tm)
    nt = Mp // tm
    xp = jnp.pad(xm, ((0, Mp - M), (0, 0), (0, 0)))
    xr = xp.reshape(Mp, H // 2, 2, W).transpose(0, 2, 1, 3)

    stats = pl.pallas_call(
        _gram_kernel,
        out_shape=jax.ShapeDtypeStruct((nt, K3, K3), jnp.float32),
        grid=(nt,),
        in_specs=[pl.BlockSpec((tm, 2, H // 2, W), lambda i: (i, 0, 0, 0))],
        out_specs=pl.BlockSpec((1, K3, K3), lambda i: (i, 0, 0)),
        compiler_params=pltpu.CompilerParams(
            dimension_semantics=("parallel",),
            vmem_limit_bytes=_VMEM_LIMIT),
    )(xr)
    GG = jnp.sum(stats, axis=0)
    colsum = GG[K3 - 1, :K3 - 1]

    idxS = np.zeros((9, W), np.int32)
    mskS = np.zeros((9, W), np.float32)
    for ky in range(3):
        for kx in range(3):
            k = ky * 3 + kx
            for w in range(W):
                wp = w + kx - 1
                if 0 <= wp < W:
                    idxS[k, w] = ky * W + wp
                    mskS[k, w] = 1.0
    S = jnp.sum(colsum[idxS] * mskS, axis=1)

    idxR = np.zeros((81, W), np.int32)
    idxC = np.zeros((81, W), np.int32)
    mskG = np.zeros((81, W), np.float32)
    for k in range(9):
        ky_k, kx_k = divmod(k, 3)
        for l in range(9):
            ky_l, kx_l = divmod(l, 3)
            for w in range(W):
                wk, wl = w + kx_k - 1, w + kx_l - 1
                if 0 <= wk < W and 0 <= wl < W:
                    idxR[k * 9 + l, w] = ky_k * W + wk
                    idxC[k * 9 + l, w] = ky_l * W + wl
                    mskG[k * 9 + l, w] = 1.0
    G = jnp.sum(GG[idxR, idxC] * mskG, axis=1).reshape(9, 9)

    count = float(M * H * W)
    mean = jnp.dot(S, w9) / count
    ssq = jnp.einsum("kc,kl,lc->c", w9, G, w9)
    var = jnp.maximum(ssq / count - mean * mean, 0.0)
    scale = gamma * lax.rsqrt(var + _EPS)
    shift = beta - scale * mean
    pool_inv = 1.0 / ((H // 2) * (W // 2))
    sf = scale * pool_inv
    hf = shift * pool_inv

    rows_l, cbase_l, taps_l = [], [], []
    for ky in range(3):
        for kx in range(3):
            for w in range(W):
                wp = w + kx - 1
                if 0 <= wp < W:
                    rows_l.append(ky * W + wp)
                    cbase_l.append((w % 2) * (W // 2) * C + (w // 2) * C)
                    taps_l.append(ky * 3 + kx)
    E = len(rows_l)
    rows_f = np.repeat(np.array(rows_l, np.int32), C)
    cols_f = np.repeat(np.array(cbase_l, np.int32), C) + np.tile(np.arange(C, dtype=np.int32), E)
    w9s = w9 * sf[None, :]
    vals = w9s[np.array(taps_l, np.int32)].reshape(-1)
    B = jnp.zeros((K3, W * C), jnp.float32).at[rows_f, cols_f].set(vals)
    B = B.at[K3 - 1, :].set(jnp.tile(hf, W))

    vecs = jnp.stack([fc1_b, fc2_w.reshape(-1),
                      jnp.full((C,), fc2_b[0], jnp.float32)], axis=0)

    scores = pl.pallas_call(
        _main_kernel,
        out_shape=jax.ShapeDtypeStruct((nt, tm, 1), jnp.float32),
        grid=(nt,),
        in_specs=[pl.BlockSpec((tm, 2, H // 2, W), lambda i: (i, 0, 0, 0)),
                  pl.BlockSpec((K3, W * C), lambda i: (0, 0)),
                  pl.BlockSpec((C, C), lambda i: (0, 0)),
                  pl.BlockSpec((3, C), lambda i: (0, 0))],
        out_specs=pl.BlockSpec((1, tm, 1), lambda i: (i, 0, 0)),
        compiler_params=pltpu.CompilerParams(
            dimension_semantics=("parallel",),
            vmem_limit_bytes=_VMEM_LIMIT),
    )(xr, B, fc1_w, vecs)

    return scores.reshape(Mp, 1)[:M].reshape(d0 * d1, J, 1)


# speedup vs baseline: 17.3104x; 1.0768x over previous
import numpy as np

import jax
import jax.numpy as jnp
from jax import lax
from jax.experimental import pallas as pl
from jax.experimental.pallas import tpu as pltpu

_EPS = 1e-5
_TM = 512
_VMEM_LIMIT = 60 * 1024 * 1024


def _round_up(x, k):
    return (x + k - 1) // k * k


def _build_x3(xt, tm, H, W):
    h2 = H // 2
    xe = xt[:, 0]
    xo = xt[:, 1]
    z = jnp.zeros((tm, 1, W), jnp.float32)
    dn = jnp.concatenate([z, xo[:, :h2 - 1]], axis=1)
    up = jnp.concatenate([xe[:, 1:], z], axis=1)

    def sec(a, b):
        return jnp.concatenate([a[:, None], b[:, None]], axis=1).reshape(tm * H, W)

    s0 = sec(dn, xe)
    s1 = xt.reshape(tm * H, W)
    s2 = sec(xo, up)
    ones = jnp.ones((tm * H, 1), jnp.float32)
    return jnp.concatenate([s0, s1, s2, ones], axis=1)


def _gram_kernel(xt_ref, out_ref):
    tm, _, h2, W = xt_ref.shape
    H = 2 * h2
    x3 = _build_x3(xt_ref[...], tm, H, W)
    gg = lax.dot_general(x3, x3, (((0,), (0,)), ((), ())),
                         preferred_element_type=jnp.float32)
    out_ref[...] = gg[None]


def _main_kernel(xt_ref, b_ref, fc1_ref, vec_ref, out_ref):
    tm, _, h2, W = xt_ref.shape
    H = 2 * h2
    C = fc1_ref.shape[0]
    w2 = W // 2
    x3 = _build_x3(xt_ref[...], tm, H, W)
    y = jnp.dot(x3, b_ref[...], preferred_element_type=jnp.float32)
    y = y.reshape(tm, 2, h2, W * C)
    v = jnp.maximum(y[:, 0], y[:, 1]).reshape(tm * h2, W * C)
    half = w2 * C
    z = jnp.maximum(jnp.maximum(v[:, :half], v[:, half:]), 0.0)
    while z.shape[1] > C:
        hw = z.shape[1] // 2
        z = z[:, :hw] + z[:, hw:]
    feat = jnp.sum(z.reshape(tm, h2, C), axis=1)
    vecs = vec_ref[...]
    h = jnp.dot(feat, fc1_ref[...], preferred_element_type=jnp.float32) + vecs[0:1, :]
    h = jnp.maximum(h, 0.0)
    logit = jnp.sum(h * vecs[1:2, :], axis=-1, keepdims=True) + vecs[2:3, 0:1]
    out_ref[...] = (1.0 / (1.0 + jnp.exp(-logit))).reshape(1, tm, 1)


def kernel(x, conv_w, conv_b, gamma, beta, fc1_w, fc1_b, fc2_w, fc2_b):
    d0, d1, J, H, W = x.shape
    assert H % 2 == 0 and W % 2 == 0
    M = d0 * d1 * J
    C = conv_w.shape[-1]
    K3 = 3 * W + 1

    xm = x.reshape(M, H, W).astype(jnp.float32)
    w9 = conv_w.reshape(9, C).astype(jnp.float32)

    tm = min(_TM, _round_up(M, 8))
    Mp = _round_up(M, tm)
    nt = Mp // tm
    xp = jnp.pad(xm, ((0, Mp - M), (0, 0), (0, 0)))
    xr = xp.reshape(Mp, H // 2, 2, W).transpose(0, 2, 1, 3)

    stats = pl.pallas_call(
        _gram_kernel,
        out_shape=jax.ShapeDtypeStruct((nt, K3, K3), jnp.float32),
        grid=(nt,),
        in_specs=[pl.BlockSpec((tm, 2, H // 2, W), lambda i: (i, 0, 0, 0))],
        out_specs=pl.BlockSpec((1, K3, K3), lambda i: (i, 0, 0)),
        compiler_params=pltpu.CompilerParams(
            dimension_semantics=("parallel",),
            vmem_limit_bytes=_VMEM_LIMIT),
    )(xr)
    GG = jnp.sum(stats, axis=0)

    selS = np.zeros((9, K3), np.float32)
    selG = np.zeros((81, K3, K3), np.float32)
    for k in range(9):
        ky_k, kx_k = divmod(k, 3)
        for w in range(W):
            wk = w + kx_k - 1
            if 0 <= wk < W:
                selS[k, ky_k * W + wk] += 1.0
        for l in range(9):
            ky_l, kx_l = divmod(l, 3)
            for w in range(W):
                wk, wl = w + kx_k - 1, w + kx_l - 1
                if 0 <= wk < W and 0 <= wl < W:
                    selG[k * 9 + l, ky_k * W + wk, ky_l * W + wl] += 1.0
    S = jnp.einsum("r,pr->p", GG[K3 - 1], selS)
    G = jnp.einsum("rc,prc->p", GG, selG).reshape(9, 9)

    count = float(M * H * W)
    mean = jnp.dot(S, w9) / count
    ssq = jnp.einsum("kc,kl,lc->c", w9, G, w9)
    var = jnp.maximum(ssq / count - mean * mean, 0.0)
    scale = gamma * lax.rsqrt(var + _EPS)
    shift = beta - scale * mean
    pool_inv = 1.0 / ((H // 2) * (W // 2))
    sf = scale * pool_inv
    hf = shift * pool_inv

    place = np.zeros((9, K3 - 1, W), np.float32)
    for ky in range(3):
        for kx in range(3):
            for w in range(W):
                wp = w + kx - 1
                if 0 <= wp < W:
                    place[ky * 3 + kx, ky * W + wp, w] = 1.0
    w9s = w9 * sf[None, :]
    Bwc = jnp.einsum("trw,tc->rwc", place, w9s)
    Bmain = Bwc.reshape(K3 - 1, W // 2, 2, C).transpose(0, 2, 1, 3).reshape(K3 - 1, W * C)
    shift_row = jnp.broadcast_to(hf[None, :], (W, C)).reshape(1, W * C)
    B = jnp.concatenate([Bmain, shift_row], axis=0)

    vecs = jnp.stack([fc1_b, fc2_w.reshape(-1),
                      jnp.full((C,), fc2_b[0], jnp.float32)], axis=0)

    scores = pl.pallas_call(
        _main_kernel,
        out_shape=jax.ShapeDtypeStruct((nt, tm, 1), jnp.float32),
        grid=(nt,),
        in_specs=[pl.BlockSpec((tm, 2, H // 2, W), lambda i: (i, 0, 0, 0)),
                  pl.BlockSpec((K3, W * C), lambda i: (0, 0)),
                  pl.BlockSpec((C, C), lambda i: (0, 0)),
                  pl.BlockSpec((3, C), lambda i: (0, 0))],
        out_specs=pl.BlockSpec((1, tm, 1), lambda i: (i, 0, 0)),
        compiler_params=pltpu.CompilerParams(
            dimension_semantics=("parallel",),
            vmem_limit_bytes=_VMEM_LIMIT),
    )(xr, B, fc1_w, vecs)

    return scores.reshape(Mp, 1)[:M].reshape(d0 * d1, J, 1)
